# Initial kernel scaffold; baseline (speedup 1.0000x reference)
import jax
import jax.numpy as jnp
from jax.experimental import pallas as pl


def kernel(node_attrs, edge_index, edge_vector, edge_length, W_node, Wr1, Wr2, Wl, Wsk, Wsp, Wp2):
    raise NotImplementedError("write your pallas kernel here")


# speedup vs baseline: 20.6565x; 20.6565x over previous
import functools

import jax
import jax.numpy as jnp
from jax import lax
from jax.experimental import pallas as pl
from jax.experimental.pallas import tpu as pltpu
from jax.experimental.pallas import tpu_sc as plsc

N = 10000
E = 320000
C = 64
NSP = 4
NB = 8
NSH = 9
RC = 5.0
AVG = 32.0

NC = 2
NS = 16
NW = NC * NS
EPW = E // NW
B = 80
K = EPW // B
NPAD = 10240


def _sc_gather_body(h_hbm, src_hbm, out_hbm, idx_v, rows0, rows1, sem0, sem1):
    wid = lax.axis_index("c") * NS + lax.axis_index("s")
    base = wid * EPW
    pltpu.sync_copy(src_hbm.at[wid], idx_v)

    pltpu.async_copy(h_hbm.at[idx_v.at[0]], rows0, sem0)

    def body(j, _):
        even = lax.rem(j, 2) == 0
        @pl.when(even)
        def _():
            pltpu.async_copy(h_hbm.at[idx_v.at[j + 1]], rows1, sem1)
            pltpu.make_async_copy(h_hbm.at[idx_v.at[j]], rows0, sem0).wait()
            pltpu.sync_copy(rows0, out_hbm.at[pl.ds(base + j * B, B)])
        @pl.when(jnp.logical_not(even))
        def _():
            pltpu.async_copy(h_hbm.at[idx_v.at[j + 1]], rows0, sem0)
            pltpu.make_async_copy(h_hbm.at[idx_v.at[j]], rows1, sem1).wait()
            pltpu.sync_copy(rows1, out_hbm.at[pl.ds(base + j * B, B)])
        return _

    lax.fori_loop(0, K - 1, body, None)
    last_even = (K - 1) % 2 == 0
    last_rows, last_sem = (rows0, sem0) if last_even else (rows1, sem1)
    pltpu.make_async_copy(h_hbm.at[idx_v.at[K - 1]], last_rows, last_sem).wait()
    pltpu.sync_copy(last_rows, out_hbm.at[pl.ds(base + (K - 1) * B, B)])


def _sc_scatter_body(t_hbm, dst_hbm, z_hbm, out_hbm, idx_v, rows0, rows1, acc,
                     sem0, sem1):
    cid = lax.axis_index("c")
    sid = lax.axis_index("s")
    wid = cid * NS + sid
    base = wid * EPW
    rps = NPAD // NS

    pltpu.sync_copy(z_hbm.at[pl.ds(sid * rps, rps)], acc.at[pl.ds(sid * rps, rps)])
    pltpu.sync_copy(dst_hbm.at[wid], idx_v)
    plsc.subcore_barrier()

    pltpu.async_copy(t_hbm.at[pl.ds(base, B)], rows0, sem0)

    def body(j, _):
        even = lax.rem(j, 2) == 0
        @pl.when(even)
        def _():
            pltpu.async_copy(t_hbm.at[pl.ds(base + (j + 1) * B, B)], rows1, sem1)
            pltpu.make_async_copy(t_hbm.at[pl.ds(base + j * B, B)], rows0, sem0).wait()
            pltpu.sync_copy(rows0, acc.at[idx_v.at[j]], add=True)
        @pl.when(jnp.logical_not(even))
        def _():
            pltpu.async_copy(t_hbm.at[pl.ds(base + (j + 1) * B, B)], rows0, sem0)
            pltpu.make_async_copy(t_hbm.at[pl.ds(base + j * B, B)], rows1, sem1).wait()
            pltpu.sync_copy(rows1, acc.at[idx_v.at[j]], add=True)
        return _

    lax.fori_loop(0, K - 1, body, None)
    last_even = (K - 1) % 2 == 0
    last_rows, last_sem = (rows0, sem0) if last_even else (rows1, sem1)
    pltpu.make_async_copy(
        t_hbm.at[pl.ds(base + (K - 1) * B, B)], last_rows, last_sem).wait()
    pltpu.sync_copy(last_rows, acc.at[idx_v.at[K - 1]], add=True)

    plsc.subcore_barrier()
    pltpu.sync_copy(acc.at[pl.ds(sid * rps, rps)],
                    out_hbm.at[cid, pl.ds(sid * rps, rps)])


@functools.lru_cache(maxsize=None)
def _sc_kernels():
    mesh = plsc.VectorSubcoreMesh(core_axis_name="c", subcore_axis_name="s",
                                  num_cores=NC, num_subcores=NS)
    params = pltpu.CompilerParams(use_tc_tiling_on_sc=False)
    gather = pl.kernel(
        _sc_gather_body,
        compiler_params=params,
        out_type=jax.ShapeDtypeStruct((E, C), jnp.float32),
        mesh=mesh,
        scratch_types=[
            pltpu.VMEM((K, B), jnp.int32),
            pltpu.VMEM((B, C), jnp.float32),
            pltpu.VMEM((B, C), jnp.float32),
            pltpu.SemaphoreType.DMA,
            pltpu.SemaphoreType.DMA,
        ],
    )
    scatter = pl.kernel(
        _sc_scatter_body,
        compiler_params=params,
        out_type=jax.ShapeDtypeStruct((NC, NPAD, C), jnp.float32),
        mesh=mesh,
        scratch_types=[
            pltpu.VMEM((K, B), jnp.int32),
            pltpu.VMEM((B, C), jnp.float32),
            pltpu.VMEM((B, C), jnp.float32),
            pltpu.VMEM_SHARED((NPAD, C), jnp.float32),
            pltpu.SemaphoreType.DMA,
            pltpu.SemaphoreType.DMA,
        ],
    )
    return gather, scatter


BE = 2560
BN = 2000

_S3 = 3.0 ** 0.5
_S5 = 5.0 ** 0.5
_S15 = 15.0 ** 0.5


def _edge_body(el_ref, ev_ref, xs_ref, wr1_ref, wr2_ref, wl_ref, out_ref):
    r = el_ref[...]
    u = r * (1.0 / RC)
    u2 = u * u
    u5 = u2 * u2 * u
    fc = 1.0 - 21.0 * u5 + 35.0 * u5 * u - 15.0 * u5 * u2
    fc = fc * (u < 1.0).astype(jnp.float32)
    n = lax.broadcasted_iota(jnp.int32, (1, NB), 1).astype(jnp.float32) + 1.0
    arg = n * (jnp.pi / RC) * r
    eb = (jnp.sqrt(2.0 / RC) * fc / r) * jnp.sin(arg)

    z1 = jnp.dot(eb, wr1_ref[...], preferred_element_type=jnp.float32)
    a1 = z1 * jax.nn.sigmoid(z1)
    rad = jnp.dot(a1, wr2_ref[...], preferred_element_type=jnp.float32)

    v = ev_ref[...]
    inv = 1.0 / (jnp.sqrt(jnp.sum(v * v, axis=1, keepdims=True)) + 1e-9)
    x = v[:, 0:1] * inv
    y = v[:, 1:2] * inv
    z = v[:, 2:3] * inv

    u_e = xs_ref[...] * rad
    vv = jnp.dot(u_e, wl_ref[...], preferred_element_type=jnp.float32)

    sh = (
        None,
        _S3 * x, _S3 * y, _S3 * z,
        _S15 * x * y, _S15 * y * z,
        0.5 * _S5 * (3.0 * z * z - 1.0),
        _S15 * x * z,
        0.5 * _S15 * (x * x - y * y),
    )
    acc = vv[:, 0:C]
    for s in range(1, NSH):
        acc = acc + vv[:, s * C:(s + 1) * C] * sh[s]
    out_ref[...] = acc


def _tc_edge(edge_length, edge_vector, xs, wr1, wr2, wl2d):
    return pl.pallas_call(
        _edge_body,
        grid=(E // BE,),
        in_specs=[
            pl.BlockSpec((BE, 1), lambda i: (i, 0)),
            pl.BlockSpec((BE, 3), lambda i: (i, 0)),
            pl.BlockSpec((BE, C), lambda i: (i, 0)),
            pl.BlockSpec((NB, C), lambda i: (0, 0)),
            pl.BlockSpec((C, C), lambda i: (0, 0)),
            pl.BlockSpec((C, NSH * C), lambda i: (0, 0)),
        ],
        out_specs=pl.BlockSpec((BE, C), lambda i: (i, 0)),
        out_shape=jax.ShapeDtypeStruct((E, C), jnp.float32),
    )(edge_length, edge_vector, xs, wr1, wr2, wl2d)


def _onehot_mix(na, w):
    return sum(na[:, k:k + 1] * w[k:k + 1, :] for k in range(NSP))


def _embed_body(na_ref, w_ref, out_ref):
    out_ref[...] = _onehot_mix(na_ref[...], w_ref[...])


def _tc_embed(node_attrs, w_node):
    return pl.pallas_call(
        _embed_body,
        grid=(N // BN,),
        in_specs=[
            pl.BlockSpec((BN, NSP), lambda i: (i, 0)),
            pl.BlockSpec((NSP, C), lambda i: (0, 0)),
        ],
        out_specs=pl.BlockSpec((BN, C), lambda i: (i, 0)),
        out_shape=jax.ShapeDtypeStruct((N, C), jnp.float32),
    )(node_attrs, w_node)


def _node_body(p_ref, h_ref, na_ref, wsp_ref, wsk_ref, wp2_ref, out_ref):
    m = (p_ref[0] + p_ref[1]) * (1.0 / AVG)
    na = na_ref[...]
    sp = _onehot_mix(na, wsp_ref[...])
    sk = _onehot_mix(na, wsk_ref[...])
    quad = jnp.dot(m * m, wp2_ref[...], preferred_element_type=jnp.float32)
    out_ref[...] = m * sp + quad + h_ref[...] * sk


def _tc_node(partials, h, node_attrs, wsp, wsk, wp2):
    return pl.pallas_call(
        _node_body,
        grid=(N // BN,),
        in_specs=[
            pl.BlockSpec((NC, BN, C), lambda i: (0, i, 0)),
            pl.BlockSpec((BN, C), lambda i: (i, 0)),
            pl.BlockSpec((BN, NSP), lambda i: (i, 0)),
            pl.BlockSpec((NSP, C), lambda i: (0, 0)),
            pl.BlockSpec((NSP, C), lambda i: (0, 0)),
            pl.BlockSpec((C, C), lambda i: (0, 0)),
        ],
        out_specs=pl.BlockSpec((BN, C), lambda i: (i, 0)),
        out_shape=jax.ShapeDtypeStruct((N, C), jnp.float32),
    )(partials, h, node_attrs, wsp, wsk, wp2)


def kernel(node_attrs, edge_index, edge_vector, edge_length,
           W_node, Wr1, Wr2, Wl, Wsk, Wsp, Wp2):
    src = edge_index[0].astype(jnp.int32).reshape(NW, K, B)
    dst = edge_index[1].astype(jnp.int32).reshape(NW, K, B)
    zeros = jnp.zeros((NPAD, C), jnp.float32)

    sc_gather, sc_scatter = _sc_kernels()
    h = _tc_embed(node_attrs, W_node)
    descriptors = []
    for l in range(2):
        xs = sc_gather(h, src)
        t = _tc_edge(edge_length, edge_vector, xs,
                     Wr1[l], Wr2[l], Wl[l].reshape(C, NSH * C))
        partials = sc_scatter(t, dst, zeros)
        h = _tc_node(partials, h, node_attrs, Wsp[l], Wsk[l], Wp2[l])
        descriptors.append(h)
    return jnp.concatenate(descriptors, axis=-1)
